# baseline (speedup 1.0000x reference)
import functools

import jax
import jax.numpy as jnp
from jax import lax
from jax.experimental import pallas as pl
from jax.experimental.pallas import tpu as pltpu
from jax.experimental.pallas import tpu_sc as plsc

N = 131072
C = 190
NC = 2
NS = 16
L = 16
NW = NC * NS
BPW = N // NW
IGNORE = -1


def _sc_body(logits_hbm, tgt_hbm, sums_hbm, cnts_hbm, idx_v, flat_v, val_v,
             acc_v, cnt_v, sem):
    c = lax.axis_index("c")
    s = lax.axis_index("s")
    wid = s * NC + c
    base = wid * BPW

    pltpu.sync_copy(tgt_hbm.at[pl.ds(base, BPW)], idx_v)

    lanes = lax.iota(jnp.int32, L)

    def mk_idx(j, _):
        t = idx_v[pl.ds(j * L, L)]
        keep = t != IGNORE
        safe = jnp.where(keep, t, 0)
        rows = base + j * L + lanes
        flat_v[pl.ds(j * L, L)] = rows * C + safe
        return 0

    lax.fori_loop(0, BPW // L, mk_idx, 0, unroll=4)

    pltpu.async_copy(logits_hbm.at[flat_v], val_v, sem).wait()

    def accum(j, carry):
        acc, cnt = carry
        x = val_v[pl.ds(j * L, L)]
        t = idx_v[pl.ds(j * L, L)]
        keep = t != IGNORE
        d = 1.0 - x
        acc = acc + jnp.where(keep, d * d, 0.0)
        cnt = cnt + jnp.where(keep, 1.0, 0.0)
        return acc, cnt

    zero = jnp.zeros((L,), jnp.float32)
    acc, cnt = lax.fori_loop(0, BPW // L, accum, (zero, zero), unroll=4)

    acc_v[...] = acc
    cnt_v[...] = cnt
    pltpu.sync_copy(acc_v, sums_hbm.at[wid])
    pltpu.sync_copy(cnt_v, cnts_hbm.at[wid])


@jax.jit
def _ppd_loss(logits_flat, tgt):
    mesh = plsc.VectorSubcoreMesh(core_axis_name="c", subcore_axis_name="s")
    sums, cnts = pl.kernel(
        _sc_body,
        out_type=[
            jax.ShapeDtypeStruct((NW, L), jnp.float32),
            jax.ShapeDtypeStruct((NW, L), jnp.float32),
        ],
        mesh=mesh,
        scratch_types=[
            pltpu.VMEM((BPW,), jnp.int32),
            pltpu.VMEM((BPW,), jnp.int32),
            pltpu.VMEM((BPW,), jnp.float32),
            pltpu.VMEM((L,), jnp.float32),
            pltpu.VMEM((L,), jnp.float32),
            pltpu.SemaphoreType.DMA,
        ],
    )(logits_flat, tgt)
    return jnp.sum(sums) / jnp.sum(cnts)


def kernel(contrast_logits, contrast_target):
    return _ppd_loss(contrast_logits.reshape(-1), contrast_target)


# speedup vs baseline: 1.1447x; 1.1447x over previous
import functools

import jax
import jax.numpy as jnp
from jax import lax
from jax.experimental import pallas as pl
from jax.experimental.pallas import tpu as pltpu

N = 131072
C = 190
BR = 1024
NB = N // BR
IGNORE = -1


def _tc_body(tgt_ref, logits_ref, out_ref):
    x = logits_ref[...]
    t = tgt_ref[0, 0, :]
    t2 = t.reshape(BR, 1)
    keep = t2 != IGNORE
    cols = lax.broadcasted_iota(jnp.int32, (BR, C), 1)
    mask = (cols == t2) & keep
    d = 1.0 - x
    ssum = jnp.sum(jnp.where(mask, d * d, 0.0))
    cnt = jnp.sum(keep.astype(jnp.float32))
    lane = lax.broadcasted_iota(jnp.int32, (1, 1, 128), 2)
    out_ref[...] = jnp.where(lane == 0, ssum, jnp.where(lane == 1, cnt, 0.0))


@jax.jit
def _ppd_loss(logits, tgt):
    tgt3 = tgt.reshape(NB, 1, BR)
    out = pl.pallas_call(
        _tc_body,
        grid=(NB,),
        in_specs=[
            pl.BlockSpec((1, 1, BR), lambda b: (b, 0, 0)),
            pl.BlockSpec((BR, C), lambda b: (b, 0)),
        ],
        out_specs=pl.BlockSpec((1, 1, 128), lambda b: (b, 0, 0)),
        out_shape=jax.ShapeDtypeStruct((NB, 1, 128), jnp.float32),
        compiler_params=pltpu.CompilerParams(
            dimension_semantics=("arbitrary",),
        ),
    )(tgt3, logits)
    return jnp.sum(out[:, 0, 0]) / jnp.sum(out[:, 0, 1])


def kernel(contrast_logits, contrast_target):
    return _ppd_loss(contrast_logits, contrast_target)


# speedup vs baseline: 1.5132x; 1.3219x over previous
import functools

import jax
import jax.numpy as jnp
from jax import lax
from jax.experimental import pallas as pl
from jax.experimental.pallas import tpu as pltpu

N = 131072
C = 190
BR = 4096
NB = N // BR
IGNORE = -1


def _tc_body(tgt_ref, logits_ref, out_ref):
    x = logits_ref[...]
    t = tgt_ref[0, 0, :]
    t2 = t.reshape(BR, 1)
    cols = lax.broadcasted_iota(jnp.int32, (BR, C), 1)
    mask = cols == t2
    d = 1.0 - x
    ssum = jnp.sum(jnp.where(mask, d * d, 0.0))
    cnt = jnp.sum((tgt_ref[...] != IGNORE).astype(jnp.float32))
    lane = lax.broadcasted_iota(jnp.int32, (1, 1, 128), 2)
    out_ref[...] = jnp.where(lane == 0, ssum, jnp.where(lane == 1, cnt, 0.0))


@jax.jit
def _ppd_loss(logits, tgt):
    tgt3 = tgt.reshape(NB, 1, BR)
    out = pl.pallas_call(
        _tc_body,
        grid=(NB,),
        in_specs=[
            pl.BlockSpec((1, 1, BR), lambda b: (b, 0, 0)),
            pl.BlockSpec((BR, C), lambda b: (b, 0)),
        ],
        out_specs=pl.BlockSpec((1, 1, 128), lambda b: (b, 0, 0)),
        out_shape=jax.ShapeDtypeStruct((NB, 1, 128), jnp.float32),
        compiler_params=pltpu.CompilerParams(
            dimension_semantics=("arbitrary",),
        ),
    )(tgt3, logits)
    return jnp.sum(out[:, 0, 0]) / jnp.sum(out[:, 0, 1])


def kernel(contrast_logits, contrast_target):
    return _ppd_loss(contrast_logits, contrast_target)
